# baseline (speedup 1.0000x reference)
import functools

import jax
import jax.numpy as jnp
from jax import lax
from jax.experimental import pallas as pl
from jax.experimental.pallas import tpu as pltpu
from jax.experimental.pallas import tpu_sc as plsc

AGG = 0.3
N_NODES = 10000
N_PAD = 10240
D = 256
DH = 128
N_EDGES = 160000
CHUNK = 48
NBUF = 3
BODY_E = NBUF * CHUNK
N_BODIES = 72
E_TILE = N_BODIES * BODY_E
E_PAD = 16 * E_TILE
EPK = 3 * CHUNK
BODY_W = NBUF * EPK
NODES_TILE = N_PAD // 16
NODE_CHUNK = 32
N_NODE_CHUNKS = NODES_TILE // NODE_CHUNK

_mesh = plsc.VectorSubcoreMesh(core_axis_name="c", subcore_axis_name="s")


@functools.partial(
    pl.kernel,
    mesh=_mesh,
    out_type=jax.ShapeDtypeStruct((2 * N_PAD, DH), jnp.float32),
    scratch_types=[
        pltpu.VMEM((BODY_W,), jnp.int32),
        pltpu.VMEM((NBUF, CHUNK), jnp.int32),
        pltpu.VMEM((NBUF, CHUNK), jnp.int32),
        pltpu.VMEM((NBUF, CHUNK), jnp.float32),
        pltpu.VMEM((CHUNK, DH), jnp.float32),
        pltpu.VMEM((CHUNK, DH), jnp.float32),
        pltpu.VMEM((CHUNK, DH), jnp.float32),
        pltpu.VMEM((N_PAD,), jnp.float32),
        pltpu.VMEM((NODES_TILE,), jnp.float32),
        pltpu.VMEM((NODES_TILE,), jnp.float32),
        pltpu.VMEM_SHARED((N_PAD, DH), jnp.float32),
        pltpu.VMEM_SHARED((16, N_PAD), jnp.float32),
        pltpu.SemaphoreType.DMA,
        pltpu.SemaphoreType.DMA,
        pltpu.SemaphoreType.DMA,
        pltpu.SemaphoreType.DMA,
        pltpu.SemaphoreType.DMA,
        pltpu.SemaphoreType.DMA,
        pltpu.SemaphoreType.DMA,
    ],
    compiler_params=pltpu.CompilerParams(needs_layout_passes=False),
)
def _sc_body(ftab, edata_h, out_h,
             ebuf, idx, tgt, wsv, rows0, rows1, rows2,
             counts_v, a_ref, b_ref, enh_sh, counts_sh,
             sem_e, sem_g0, sem_g1, sem_g2, sem_s0, sem_s1, sem_s2):
    c = lax.axis_index("c")
    s = lax.axis_index("s")
    zero16 = jnp.zeros((16,), jnp.float32)
    rows = [rows0, rows1, rows2]
    sem_g = [sem_g0, sem_g1, sem_g2]
    sem_s = [sem_s0, sem_s1, sem_s2]

    def _zc(i, _):
        counts_v[pl.ds(i * 16, 16)] = zero16
        return 0
    lax.fori_loop(0, N_PAD // 16, _zc, 0)

    def _zr(i, _):
        for k in range(DH // 16):
            for r in rows:
                r[i, pl.ds(k * 16, 16)] = zero16
        return 0
    lax.fori_loop(0, CHUNK, _zr, 0)

    nbase = s * NODES_TILE
    def _zs(j, _):
        pltpu.sync_copy(rows0.at[pl.ds(0, NODE_CHUNK)],
                        enh_sh.at[pl.ds(nbase + j * NODE_CHUNK, NODE_CHUNK)])
        return 0
    lax.fori_loop(0, NODES_TILE // NODE_CHUNK, _zs, 0)

    plsc.subcore_barrier()

    ebase = s * N_BODIES * BODY_W
    coff = c * N_PAD

    def _prep(u):
        o = u * EPK
        for q in range(CHUNK // 16):
            idx[u, pl.ds(q * 16, 16)] = ebuf[pl.ds(o + q * 16, 16)] + coff
            tgt[u, pl.ds(q * 16, 16)] = ebuf[pl.ds(o + CHUNK + q * 16, 16)]
            wsv[u, pl.ds(q * 16, 16)] = plsc.bitcast(
                ebuf[pl.ds(o + 2 * CHUNK + q * 16, 16)], jnp.float32)

    def _scale_counts(u):
        rows_u = rows[u]

        def _scale(q, _):
            wg = wsv[u, pl.ds(q * 16, 16)]
            for l in range(16):
                i = q * 16 + l
                ws = wg[l]
                for k in range(DH // 16):
                    rows_u[i, pl.ds(k * 16, 16)] = rows_u[i, pl.ds(k * 16, 16)] * ws
            return 0
        lax.fori_loop(0, CHUNK // 16, _scale, 0)
        for q in range(CHUNK // 16):
            plsc.addupdate_scatter(
                counts_v,
                [tgt[u, pl.ds(q * 16, 16)]],
                wsv[u, pl.ds(q * 16, 16)],
            )

    pltpu.async_copy(edata_h.at[pl.ds(ebase, BODY_W)], ebuf, sem_e)

    def _body(it, _):
        pltpu.make_async_copy(edata_h.at[pl.ds(ebase, BODY_W)], ebuf, sem_e).wait()
        for u in range(NBUF):
            @pl.when(it > 0)
            def _():
                pltpu.make_async_copy(rows[u], enh_sh.at[tgt.at[u]], sem_s[u]).wait()
            _prep(u)
            pltpu.async_copy(ftab.at[idx.at[u]], rows[u], sem_g[u])
            if u == NBUF - 1:
                nxt = ebase + jnp.minimum((it + 1) * BODY_W,
                                          (N_BODIES - 1) * BODY_W)
                pltpu.async_copy(edata_h.at[pl.ds(nxt, BODY_W)], ebuf, sem_e)
            if u >= 1:
                p = u - 1
                pltpu.make_async_copy(ftab.at[idx.at[p]], rows[p], sem_g[p]).wait()
                _scale_counts(p)
                pltpu.async_copy(rows[p], enh_sh.at[tgt.at[p]], sem_s[p], add=True)
        p = NBUF - 1
        pltpu.make_async_copy(ftab.at[idx.at[p]], rows[p], sem_g[p]).wait()
        _scale_counts(p)
        pltpu.async_copy(rows[p], enh_sh.at[tgt.at[p]], sem_s[p], add=True)
        return 0

    lax.fori_loop(0, N_BODIES, _body, 0)

    pltpu.make_async_copy(edata_h.at[pl.ds(ebase, BODY_W)], ebuf, sem_e).wait()
    for u in range(NBUF):
        pltpu.make_async_copy(rows[u], enh_sh.at[tgt.at[u]], sem_s[u]).wait()

    pltpu.sync_copy(counts_v, counts_sh.at[s])
    plsc.subcore_barrier()

    c16 = rows0.at[pl.ds(0, 16)]

    def _coef_chunk(j, _):
        pltpu.sync_copy(counts_sh.at[:, pl.ds(nbase + j * 128, 128)], c16)
        for k in range(128 // 16):
            acc = rows0[0, pl.ds(k * 16, 16)]
            for t in range(1, 16):
                acc = acc + rows0[t, pl.ds(k * 16, 16)]
            clamped = jnp.maximum(acc, 1e-8)
            am = jnp.where(acc > 1e-8, jnp.float32(AGG), jnp.float32(0.0))
            a_ref[pl.ds(j * 128 + k * 16, 16)] = 1.0 - am
            b_ref[pl.ds(j * 128 + k * 16, 16)] = am / clamped
        return 0
    lax.fori_loop(0, NODES_TILE // 128, _coef_chunk, 0)

    def _node_chunk(j, _):
        nb = nbase + j * NODE_CHUNK
        pltpu.sync_copy(enh_sh.at[pl.ds(nb, NODE_CHUNK)], rows0.at[pl.ds(0, NODE_CHUNK)])
        pltpu.sync_copy(ftab.at[pl.ds(coff + nb, NODE_CHUNK)], rows1.at[pl.ds(0, NODE_CHUNK)])

        def _blend(q, _):
            ag = a_ref[pl.ds(j * NODE_CHUNK + q * 16, 16)]
            bg = b_ref[pl.ds(j * NODE_CHUNK + q * 16, 16)]
            for l in range(16):
                i = q * 16 + l
                av = ag[l]
                bv = bg[l]
                for k in range(DH // 16):
                    rows1[i, pl.ds(k * 16, 16)] = (
                        rows1[i, pl.ds(k * 16, 16)] * av
                        + rows0[i, pl.ds(k * 16, 16)] * bv
                    )
            return 0
        lax.fori_loop(0, NODE_CHUNK // 16, _blend, 0)

        pltpu.sync_copy(rows1.at[pl.ds(0, NODE_CHUNK)],
                        out_h.at[pl.ds(coff + nb, NODE_CHUNK)])
        return 0

    lax.fori_loop(0, N_NODE_CHUNKS, _node_chunk, 0)


def kernel(features, edges, edge_weights):
    f0 = jnp.pad(features[:, :DH], ((0, N_PAD - N_NODES), (0, 0)))
    f1 = jnp.pad(features[:, DH:], ((0, N_PAD - N_NODES), (0, 0)))
    ftab = jnp.concatenate([f0, f1], axis=0)
    src = jnp.pad(edges[:, 0], (0, E_PAD - N_EDGES))
    tgt = jnp.pad(edges[:, 1], (0, E_PAD - N_EDGES))
    wb = jax.lax.bitcast_convert_type(
        jnp.pad(edge_weights, (0, E_PAD - N_EDGES)), jnp.int32)
    edata = jnp.stack(
        [src.reshape(-1, CHUNK), tgt.reshape(-1, CHUNK), wb.reshape(-1, CHUNK)],
        axis=1).reshape(-1)
    out = _sc_body(ftab, edata)
    return jnp.concatenate([out[:N_NODES], out[N_PAD:N_PAD + N_NODES]], axis=1)


# speedup vs baseline: 1.4485x; 1.4485x over previous
import functools

import jax
import jax.numpy as jnp
from jax import lax
from jax.experimental import pallas as pl
from jax.experimental.pallas import tpu as pltpu
from jax.experimental.pallas import tpu_sc as plsc

AGG = 0.3
N_NODES = 10000
N_PAD = 10240
D = 256
DH = 128
N_EDGES = 160000
E_PAD = 163840
E_TILE = E_PAD // 16
CHUNK = 64
EPK = 3 * CHUNK
BLOCK = 1024
BLOCK_W = (BLOCK // CHUNK) * EPK
N_BLOCKS = E_TILE // BLOCK
PAIRS = BLOCK // (2 * CHUNK)
NODES_TILE = N_PAD // 16
NODE_CHUNK = 64
N_NODE_CHUNKS = NODES_TILE // NODE_CHUNK

_mesh = plsc.VectorSubcoreMesh(core_axis_name="c", subcore_axis_name="s")


@functools.partial(
    pl.kernel,
    mesh=_mesh,
    out_type=jax.ShapeDtypeStruct((2 * N_PAD, DH), jnp.float32),
    scratch_types=[
        pltpu.VMEM((BLOCK_W,), jnp.int32),
        pltpu.VMEM((2, CHUNK), jnp.int32),
        pltpu.VMEM((2, CHUNK), jnp.int32),
        pltpu.VMEM((2, CHUNK), jnp.float32),
        pltpu.VMEM((CHUNK, DH), jnp.float32),
        pltpu.VMEM((CHUNK, DH), jnp.float32),
        pltpu.VMEM((N_PAD,), jnp.float32),
        pltpu.VMEM((NODES_TILE,), jnp.float32),
        pltpu.VMEM((NODES_TILE,), jnp.float32),
        pltpu.VMEM_SHARED((N_PAD, DH), jnp.float32),
        pltpu.VMEM_SHARED((16, N_PAD), jnp.float32),
        pltpu.SemaphoreType.DMA,
        pltpu.SemaphoreType.DMA,
        pltpu.SemaphoreType.DMA,
        pltpu.SemaphoreType.DMA,
    ],
    compiler_params=pltpu.CompilerParams(needs_layout_passes=False),
)
def _sc_body(ftab, edata_h, out_h,
             ebuf, idx, tgt, wsv, rows_a, rows_b, counts_v, a_ref, b_ref,
             enh_sh, counts_sh, sem_ga, sem_gb, sem_sa, sem_sb):
    c = lax.axis_index("c")
    s = lax.axis_index("s")
    zero16 = jnp.zeros((16,), jnp.float32)

    def _zc(i, _):
        counts_v[pl.ds(i * 16, 16)] = zero16
        return 0
    lax.fori_loop(0, N_PAD // 16, _zc, 0)

    def _zr(i, _):
        for k in range(DH // 16):
            rows_a[i, pl.ds(k * 16, 16)] = zero16
            rows_b[i, pl.ds(k * 16, 16)] = zero16
        return 0
    lax.fori_loop(0, CHUNK, _zr, 0)

    nbase = s * NODES_TILE
    def _zs(j, _):
        pltpu.sync_copy(rows_a, enh_sh.at[pl.ds(nbase + 2 * j * CHUNK, CHUNK)])
        pltpu.sync_copy(rows_b, enh_sh.at[pl.ds((2 * j + 1) * CHUNK + nbase, CHUNK)])
        return 0
    lax.fori_loop(0, NODES_TILE // (2 * CHUNK), _zs, 0)

    plsc.subcore_barrier()

    ebase = s * N_BLOCKS * BLOCK_W
    coff = c * N_PAD

    def _prep(o, u):
        for q in range(CHUNK // 16):
            idx[u, pl.ds(q * 16, 16)] = ebuf[pl.ds(o + q * 16, 16)] + coff
            tgt[u, pl.ds(q * 16, 16)] = ebuf[pl.ds(o + CHUNK + q * 16, 16)]
            wsv[u, pl.ds(q * 16, 16)] = plsc.bitcast(
                ebuf[pl.ds(o + 2 * CHUNK + q * 16, 16)], jnp.float32)

    def _scale_counts(u, rows_x):
        def _scale(q, _):
            wg = wsv[u, pl.ds(q * 16, 16)]
            for l in range(16):
                i = q * 16 + l
                ws = wg[l]
                for k in range(DH // 16):
                    rows_x[i, pl.ds(k * 16, 16)] = rows_x[i, pl.ds(k * 16, 16)] * ws
            return 0
        lax.fori_loop(0, CHUNK // 16, _scale, 0)
        for q in range(CHUNK // 16):
            plsc.addupdate_scatter(
                counts_v,
                [tgt[u, pl.ds(q * 16, 16)]],
                wsv[u, pl.ds(q * 16, 16)],
            )

    def _block(blk, _):
        pltpu.sync_copy(edata_h.at[pl.ds(ebase + blk * BLOCK_W, BLOCK_W)], ebuf)

        @pl.when(blk > 0)
        def _():
            pltpu.make_async_copy(rows_a, enh_sh.at[tgt.at[0]], sem_sa).wait()
        _prep(0, 0)
        pltpu.async_copy(ftab.at[idx.at[0]], rows_a, sem_ga)

        @pl.when(blk > 0)
        def _():
            pltpu.make_async_copy(rows_b, enh_sh.at[tgt.at[1]], sem_sb).wait()
        _prep(EPK, 1)
        pltpu.async_copy(ftab.at[idx.at[1]], rows_b, sem_gb)

        def _pair(p, _):
            oa = p * (2 * EPK)
            pltpu.make_async_copy(ftab.at[idx.at[0]], rows_a, sem_ga).wait()
            _scale_counts(0, rows_a)
            pltpu.async_copy(rows_a, enh_sh.at[tgt.at[0]], sem_sa, add=True)
            pltpu.make_async_copy(ftab.at[idx.at[1]], rows_b, sem_gb).wait()
            _scale_counts(1, rows_b)
            pltpu.async_copy(rows_b, enh_sh.at[tgt.at[1]], sem_sb, add=True)

            @pl.when(p < PAIRS - 1)
            def _():
                pltpu.make_async_copy(rows_a, enh_sh.at[tgt.at[0]], sem_sa).wait()
                _prep(oa + 2 * EPK, 0)
                pltpu.async_copy(ftab.at[idx.at[0]], rows_a, sem_ga)
                pltpu.make_async_copy(rows_b, enh_sh.at[tgt.at[1]], sem_sb).wait()
                _prep(oa + 3 * EPK, 1)
                pltpu.async_copy(ftab.at[idx.at[1]], rows_b, sem_gb)
            return 0

        lax.fori_loop(0, PAIRS, _pair, 0)
        return 0

    lax.fori_loop(0, N_BLOCKS, _block, 0)

    pltpu.make_async_copy(rows_a, enh_sh.at[tgt.at[0]], sem_sa).wait()
    pltpu.make_async_copy(rows_b, enh_sh.at[tgt.at[1]], sem_sb).wait()

    pltpu.sync_copy(counts_v, counts_sh.at[s])
    plsc.subcore_barrier()

    c16 = rows_a.at[pl.ds(0, 16)]

    def _coef_chunk(j, _):
        pltpu.sync_copy(counts_sh.at[:, pl.ds(nbase + j * 128, 128)], c16)
        for k in range(128 // 16):
            acc = rows_a[0, pl.ds(k * 16, 16)]
            for t in range(1, 16):
                acc = acc + rows_a[t, pl.ds(k * 16, 16)]
            clamped = jnp.maximum(acc, 1e-8)
            am = jnp.where(acc > 1e-8, jnp.float32(AGG), jnp.float32(0.0))
            a_ref[pl.ds(j * 128 + k * 16, 16)] = 1.0 - am
            b_ref[pl.ds(j * 128 + k * 16, 16)] = am / clamped
        return 0
    lax.fori_loop(0, NODES_TILE // 128, _coef_chunk, 0)

    def _node_chunk(j, _):
        nb = nbase + j * NODE_CHUNK
        pltpu.sync_copy(enh_sh.at[pl.ds(nb, NODE_CHUNK)], rows_a)
        pltpu.sync_copy(ftab.at[pl.ds(coff + nb, NODE_CHUNK)], rows_b)

        def _blend(q, _):
            ag = a_ref[pl.ds(j * NODE_CHUNK + q * 16, 16)]
            bg = b_ref[pl.ds(j * NODE_CHUNK + q * 16, 16)]
            for l in range(16):
                i = q * 16 + l
                av = ag[l]
                bv = bg[l]
                for k in range(DH // 16):
                    rows_b[i, pl.ds(k * 16, 16)] = (
                        rows_b[i, pl.ds(k * 16, 16)] * av
                        + rows_a[i, pl.ds(k * 16, 16)] * bv
                    )
            return 0
        lax.fori_loop(0, NODE_CHUNK // 16, _blend, 0)

        pltpu.sync_copy(rows_b, out_h.at[pl.ds(coff + nb, NODE_CHUNK)])
        return 0

    lax.fori_loop(0, N_NODE_CHUNKS, _node_chunk, 0)


def kernel(features, edges, edge_weights):
    f0 = jnp.pad(features[:, :DH], ((0, N_PAD - N_NODES), (0, 0)))
    f1 = jnp.pad(features[:, DH:], ((0, N_PAD - N_NODES), (0, 0)))
    ftab = jnp.concatenate([f0, f1], axis=0)
    src = jnp.pad(edges[:, 0], (0, E_PAD - N_EDGES))
    tgt = jnp.pad(edges[:, 1], (0, E_PAD - N_EDGES))
    wb = jax.lax.bitcast_convert_type(
        jnp.pad(edge_weights, (0, E_PAD - N_EDGES)), jnp.int32)
    edata = jnp.stack(
        [src.reshape(-1, CHUNK), tgt.reshape(-1, CHUNK), wb.reshape(-1, CHUNK)],
        axis=1).reshape(-1)
    out = _sc_body(ftab, edata)
    return jnp.concatenate([out[:N_NODES], out[N_PAD:N_PAD + N_NODES]], axis=1)


# speedup vs baseline: 1.5626x; 1.0788x over previous
import functools

import jax
import jax.numpy as jnp
from jax import lax
from jax.experimental import pallas as pl
from jax.experimental.pallas import tpu as pltpu
from jax.experimental.pallas import tpu_sc as plsc

AGG = 0.3
N_NODES = 10000
N_PAD = 10240
D = 256
DH = 128
N_EDGES = 160000
E_PAD = 163840
E_TILE = E_PAD // 16
CHUNK = 64
EPK = 3 * CHUNK
BLOCK = 2048
BLOCK_W = (BLOCK // CHUNK) * EPK
N_BLOCKS = E_TILE // BLOCK
PAIRS = BLOCK // (2 * CHUNK)
NODES_TILE = N_PAD // 16
NODE_CHUNK = 64
N_NODE_CHUNKS = NODES_TILE // NODE_CHUNK

_mesh = plsc.VectorSubcoreMesh(core_axis_name="c", subcore_axis_name="s")


@functools.partial(
    pl.kernel,
    mesh=_mesh,
    out_type=jax.ShapeDtypeStruct((2 * N_PAD, DH), jnp.float32),
    scratch_types=[
        pltpu.VMEM((BLOCK_W,), jnp.int32),
        pltpu.VMEM((2, CHUNK), jnp.int32),
        pltpu.VMEM((2, CHUNK), jnp.int32),
        pltpu.VMEM((2, CHUNK), jnp.float32),
        pltpu.VMEM((CHUNK, DH), jnp.float32),
        pltpu.VMEM((CHUNK, DH), jnp.float32),
        pltpu.VMEM((N_PAD,), jnp.float32),
        pltpu.VMEM((NODES_TILE,), jnp.float32),
        pltpu.VMEM((NODES_TILE,), jnp.float32),
        pltpu.VMEM_SHARED((N_PAD, DH), jnp.float32),
        pltpu.VMEM_SHARED((16, N_PAD), jnp.float32),
        pltpu.SemaphoreType.DMA,
        pltpu.SemaphoreType.DMA,
        pltpu.SemaphoreType.DMA,
        pltpu.SemaphoreType.DMA,
    ],
    compiler_params=pltpu.CompilerParams(needs_layout_passes=False),
)
def _sc_body(ftab, edata_h, out_h,
             ebuf, idx, tgt, wsv, rows_a, rows_b, counts_v, a_ref, b_ref,
             enh_sh, counts_sh, sem_ga, sem_gb, sem_sa, sem_sb):
    c = lax.axis_index("c")
    s = lax.axis_index("s")
    zero16 = jnp.zeros((16,), jnp.float32)

    def _zc(i, _):
        counts_v[pl.ds(i * 16, 16)] = zero16
        return 0
    lax.fori_loop(0, N_PAD // 16, _zc, 0)

    def _zr(i, _):
        for k in range(DH // 16):
            rows_a[i, pl.ds(k * 16, 16)] = zero16
            rows_b[i, pl.ds(k * 16, 16)] = zero16
        return 0
    lax.fori_loop(0, CHUNK, _zr, 0)

    nbase = s * NODES_TILE
    def _zs(j, _):
        pltpu.async_copy(rows_a, enh_sh.at[pl.ds(nbase + 2 * j * CHUNK, CHUNK)], sem_ga)
        pltpu.async_copy(rows_b, enh_sh.at[pl.ds((2 * j + 1) * CHUNK + nbase, CHUNK)], sem_gb)
        pltpu.make_async_copy(rows_a, enh_sh.at[pl.ds(nbase, CHUNK)], sem_ga).wait()
        pltpu.make_async_copy(rows_b, enh_sh.at[pl.ds(nbase, CHUNK)], sem_gb).wait()
        return 0
    lax.fori_loop(0, NODES_TILE // (2 * CHUNK), _zs, 0)

    plsc.subcore_barrier()

    ebase = s * N_BLOCKS * BLOCK_W
    coff = c * N_PAD

    def _prep(o, u):
        for q in range(CHUNK // 16):
            idx[u, pl.ds(q * 16, 16)] = ebuf[pl.ds(o + q * 16, 16)] + coff
            tgt[u, pl.ds(q * 16, 16)] = ebuf[pl.ds(o + CHUNK + q * 16, 16)]
            wsv[u, pl.ds(q * 16, 16)] = plsc.bitcast(
                ebuf[pl.ds(o + 2 * CHUNK + q * 16, 16)], jnp.float32)

    def _scale_counts(u, rows_x):
        def _scale(q, _):
            wg = wsv[u, pl.ds(q * 16, 16)]
            for l in range(16):
                i = q * 16 + l
                ws = wg[l]
                for k in range(DH // 16):
                    rows_x[i, pl.ds(k * 16, 16)] = rows_x[i, pl.ds(k * 16, 16)] * ws
            return 0
        lax.fori_loop(0, CHUNK // 16, _scale, 0)
        for q in range(CHUNK // 16):
            plsc.addupdate_scatter(
                counts_v,
                [tgt[u, pl.ds(q * 16, 16)]],
                wsv[u, pl.ds(q * 16, 16)],
            )

    def _block(blk, _):
        pltpu.sync_copy(edata_h.at[pl.ds(ebase + blk * BLOCK_W, BLOCK_W)], ebuf)

        @pl.when(blk > 0)
        def _():
            pltpu.make_async_copy(rows_a, enh_sh.at[tgt.at[0]], sem_sa).wait()
        _prep(0, 0)
        pltpu.async_copy(ftab.at[idx.at[0]], rows_a, sem_ga)

        def _pair(p, _):
            oa = p * (2 * EPK)

            @pl.when(blk + p > 0)
            def _():
                pltpu.make_async_copy(rows_b, enh_sh.at[tgt.at[1]], sem_sb).wait()
            _prep(oa + EPK, 1)
            pltpu.async_copy(ftab.at[idx.at[1]], rows_b, sem_gb)
            pltpu.make_async_copy(ftab.at[idx.at[0]], rows_a, sem_ga).wait()
            _scale_counts(0, rows_a)
            pltpu.async_copy(rows_a, enh_sh.at[tgt.at[0]], sem_sa, add=True)

            @pl.when(p < PAIRS - 1)
            def _():
                pltpu.make_async_copy(rows_a, enh_sh.at[tgt.at[0]], sem_sa).wait()
                _prep(oa + 2 * EPK, 0)
                pltpu.async_copy(ftab.at[idx.at[0]], rows_a, sem_ga)
            pltpu.make_async_copy(ftab.at[idx.at[1]], rows_b, sem_gb).wait()
            _scale_counts(1, rows_b)
            pltpu.async_copy(rows_b, enh_sh.at[tgt.at[1]], sem_sb, add=True)
            return 0

        lax.fori_loop(0, PAIRS, _pair, 0)
        return 0

    lax.fori_loop(0, N_BLOCKS, _block, 0)

    pltpu.make_async_copy(rows_a, enh_sh.at[tgt.at[0]], sem_sa).wait()
    pltpu.make_async_copy(rows_b, enh_sh.at[tgt.at[1]], sem_sb).wait()

    pltpu.sync_copy(counts_v, counts_sh.at[s])
    plsc.subcore_barrier()

    c16 = rows_a.at[pl.ds(0, 16)]

    def _coef_chunk(j, _):
        pltpu.sync_copy(counts_sh.at[:, pl.ds(nbase + j * 128, 128)], c16)
        for k in range(128 // 16):
            acc = rows_a[0, pl.ds(k * 16, 16)]
            for t in range(1, 16):
                acc = acc + rows_a[t, pl.ds(k * 16, 16)]
            clamped = jnp.maximum(acc, 1e-8)
            am = jnp.where(acc > 1e-8, jnp.float32(AGG), jnp.float32(0.0))
            a_ref[pl.ds(j * 128 + k * 16, 16)] = 1.0 - am
            b_ref[pl.ds(j * 128 + k * 16, 16)] = am / clamped
        return 0
    lax.fori_loop(0, NODES_TILE // 128, _coef_chunk, 0)

    def _node_chunk(j, _):
        nb = nbase + j * NODE_CHUNK
        @pl.when(j > 0)
        def _():
            pltpu.make_async_copy(rows_b, out_h.at[pl.ds(coff + nb, NODE_CHUNK)],
                                  sem_sa).wait()
        pltpu.async_copy(enh_sh.at[pl.ds(nb, NODE_CHUNK)], rows_a, sem_ga)
        pltpu.async_copy(ftab.at[pl.ds(coff + nb, NODE_CHUNK)], rows_b, sem_gb)
        pltpu.make_async_copy(enh_sh.at[pl.ds(nb, NODE_CHUNK)], rows_a, sem_ga).wait()
        pltpu.make_async_copy(ftab.at[pl.ds(coff + nb, NODE_CHUNK)], rows_b, sem_gb).wait()

        def _blend(q, _):
            ag = a_ref[pl.ds(j * NODE_CHUNK + q * 16, 16)]
            bg = b_ref[pl.ds(j * NODE_CHUNK + q * 16, 16)]
            for l in range(16):
                i = q * 16 + l
                av = ag[l]
                bv = bg[l]
                for k in range(DH // 16):
                    rows_b[i, pl.ds(k * 16, 16)] = (
                        rows_b[i, pl.ds(k * 16, 16)] * av
                        + rows_a[i, pl.ds(k * 16, 16)] * bv
                    )
            return 0
        lax.fori_loop(0, NODE_CHUNK // 16, _blend, 0)

        pltpu.async_copy(rows_b, out_h.at[pl.ds(coff + nb, NODE_CHUNK)], sem_sa)
        return 0

    lax.fori_loop(0, N_NODE_CHUNKS, _node_chunk, 0)
    pltpu.make_async_copy(rows_b, out_h.at[pl.ds(coff + nbase, NODE_CHUNK)],
                          sem_sa).wait()


def kernel(features, edges, edge_weights):
    f0 = jnp.pad(features[:, :DH], ((0, N_PAD - N_NODES), (0, 0)))
    f1 = jnp.pad(features[:, DH:], ((0, N_PAD - N_NODES), (0, 0)))
    ftab = jnp.concatenate([f0, f1], axis=0)
    src = jnp.pad(edges[:, 0], (0, E_PAD - N_EDGES))
    tgt = jnp.pad(edges[:, 1], (0, E_PAD - N_EDGES))
    wb = jax.lax.bitcast_convert_type(
        jnp.pad(edge_weights, (0, E_PAD - N_EDGES)), jnp.int32)
    edata = jnp.stack(
        [src.reshape(-1, CHUNK), tgt.reshape(-1, CHUNK), wb.reshape(-1, CHUNK)],
        axis=1).reshape(-1)
    out = _sc_body(ftab, edata)
    return jnp.concatenate([out[:N_NODES], out[N_PAD:N_PAD + N_NODES]], axis=1)


# speedup vs baseline: 1.5632x; 1.0004x over previous
import functools

import jax
import jax.numpy as jnp
from jax import lax
from jax.experimental import pallas as pl
from jax.experimental.pallas import tpu as pltpu
from jax.experimental.pallas import tpu_sc as plsc

AGG = 0.3
N_NODES = 10000
N_PAD = 10240
D = 256
DH = 128
N_EDGES = 160000
E_PAD = 163840
E_TILE = E_PAD // 16
CHUNK = 64
EPK = 3 * CHUNK
BLOCK = 2048
BLOCK_W = (BLOCK // CHUNK) * EPK
N_BLOCKS = E_TILE // BLOCK
PAIRS = BLOCK // (2 * CHUNK)
NODES_TILE = N_PAD // 16
NODE_CHUNK = 64
N_NODE_CHUNKS = NODES_TILE // NODE_CHUNK

_mesh = plsc.VectorSubcoreMesh(core_axis_name="c", subcore_axis_name="s")


@functools.partial(
    pl.kernel,
    mesh=_mesh,
    out_type=jax.ShapeDtypeStruct((2 * N_PAD, DH), jnp.float32),
    scratch_types=[
        pltpu.VMEM((BLOCK_W,), jnp.int32),
        pltpu.VMEM((CHUNK,), jnp.int32),
        pltpu.VMEM((CHUNK,), jnp.int32),
        pltpu.VMEM((CHUNK,), jnp.int32),
        pltpu.VMEM((CHUNK,), jnp.int32),
        pltpu.VMEM((CHUNK,), jnp.float32),
        pltpu.VMEM((CHUNK,), jnp.float32),
        pltpu.VMEM((CHUNK, DH), jnp.float32),
        pltpu.VMEM((CHUNK, DH), jnp.float32),
        pltpu.VMEM((N_PAD,), jnp.float32),
        pltpu.VMEM((NODES_TILE,), jnp.float32),
        pltpu.VMEM((NODES_TILE,), jnp.float32),
        pltpu.VMEM_SHARED((N_PAD, DH), jnp.float32),
        pltpu.VMEM_SHARED((16, N_PAD), jnp.float32),
        pltpu.SemaphoreType.DMA,
        pltpu.SemaphoreType.DMA,
        pltpu.SemaphoreType.DMA,
        pltpu.SemaphoreType.DMA,
    ],
    compiler_params=pltpu.CompilerParams(needs_layout_passes=False),
)
def _sc_body(ftab, edata_h, out_h,
             ebuf, idx_a, idx_b, tgt_a, tgt_b, wsv_a, wsv_b,
             rows_a, rows_b, counts_v, a_ref, b_ref,
             enh_sh, counts_sh, sem_ga, sem_gb, sem_sa, sem_sb):
    c = lax.axis_index("c")
    s = lax.axis_index("s")
    zero16 = jnp.zeros((16,), jnp.float32)

    def _zc(i, _):
        counts_v[pl.ds(i * 16, 16)] = zero16
        return 0
    lax.fori_loop(0, N_PAD // 16, _zc, 0)

    def _zr(i, _):
        for k in range(DH // 16):
            rows_a[i, pl.ds(k * 16, 16)] = zero16
            rows_b[i, pl.ds(k * 16, 16)] = zero16
        return 0
    lax.fori_loop(0, CHUNK, _zr, 0)

    nbase = s * NODES_TILE
    def _zs(j, _):
        pltpu.async_copy(rows_a, enh_sh.at[pl.ds(nbase + 2 * j * CHUNK, CHUNK)], sem_ga)
        pltpu.async_copy(rows_b, enh_sh.at[pl.ds((2 * j + 1) * CHUNK + nbase, CHUNK)], sem_gb)
        pltpu.make_async_copy(rows_a, enh_sh.at[pl.ds(nbase, CHUNK)], sem_ga).wait()
        pltpu.make_async_copy(rows_b, enh_sh.at[pl.ds(nbase, CHUNK)], sem_gb).wait()
        return 0
    lax.fori_loop(0, NODES_TILE // (2 * CHUNK), _zs, 0)

    plsc.subcore_barrier()

    ebase = s * N_BLOCKS * BLOCK_W
    coff = c * N_PAD

    def _prep(o, idx_x, tgt_x, wsv_x):
        for q in range(CHUNK // 16):
            idx_x[pl.ds(q * 16, 16)] = ebuf[pl.ds(o + q * 16, 16)] + coff
            tgt_x[pl.ds(q * 16, 16)] = ebuf[pl.ds(o + CHUNK + q * 16, 16)]
            wsv_x[pl.ds(q * 16, 16)] = plsc.bitcast(
                ebuf[pl.ds(o + 2 * CHUNK + q * 16, 16)], jnp.float32)

    def _scale_counts(wsv_x, tgt_x, rows_x):
        def _scale(q, _):
            wg = wsv_x[pl.ds(q * 16, 16)]
            for l in range(16):
                i = q * 16 + l
                ws = wg[l]
                for k in range(DH // 16):
                    rows_x[i, pl.ds(k * 16, 16)] = rows_x[i, pl.ds(k * 16, 16)] * ws
            return 0
        lax.fori_loop(0, CHUNK // 16, _scale, 0)
        for q in range(CHUNK // 16):
            plsc.addupdate_scatter(
                counts_v,
                [tgt_x[pl.ds(q * 16, 16)]],
                wsv_x[pl.ds(q * 16, 16)],
            )

    def _block(blk, _):
        pltpu.sync_copy(edata_h.at[pl.ds(ebase + blk * BLOCK_W, BLOCK_W)], ebuf)

        @pl.when(blk > 0)
        def _():
            pltpu.make_async_copy(rows_a, enh_sh.at[tgt_a], sem_sa).wait()
        _prep(0, idx_a, tgt_a, wsv_a)
        pltpu.async_copy(ftab.at[idx_a], rows_a, sem_ga)

        def _pair(p, _):
            oa = p * (2 * EPK)

            @pl.when(blk + p > 0)
            def _():
                pltpu.make_async_copy(rows_b, enh_sh.at[tgt_b], sem_sb).wait()
            _prep(oa + EPK, idx_b, tgt_b, wsv_b)
            pltpu.async_copy(ftab.at[idx_b], rows_b, sem_gb)
            pltpu.make_async_copy(ftab.at[idx_a], rows_a, sem_ga).wait()
            _scale_counts(wsv_a, tgt_a, rows_a)
            pltpu.async_copy(rows_a, enh_sh.at[tgt_a], sem_sa, add=True)

            @pl.when(p < PAIRS - 1)
            def _():
                pltpu.make_async_copy(rows_a, enh_sh.at[tgt_a], sem_sa).wait()
                _prep(oa + 2 * EPK, idx_a, tgt_a, wsv_a)
                pltpu.async_copy(ftab.at[idx_a], rows_a, sem_ga)
            pltpu.make_async_copy(ftab.at[idx_b], rows_b, sem_gb).wait()
            _scale_counts(wsv_b, tgt_b, rows_b)
            pltpu.async_copy(rows_b, enh_sh.at[tgt_b], sem_sb, add=True)
            return 0

        lax.fori_loop(0, PAIRS, _pair, 0)
        return 0

    lax.fori_loop(0, N_BLOCKS, _block, 0)

    pltpu.make_async_copy(rows_a, enh_sh.at[tgt_a], sem_sa).wait()
    pltpu.make_async_copy(rows_b, enh_sh.at[tgt_b], sem_sb).wait()

    pltpu.sync_copy(counts_v, counts_sh.at[s])
    plsc.subcore_barrier()

    c16 = rows_a.at[pl.ds(0, 16)]

    def _coef_chunk(j, _):
        pltpu.sync_copy(counts_sh.at[:, pl.ds(nbase + j * 128, 128)], c16)
        for k in range(128 // 16):
            acc = rows_a[0, pl.ds(k * 16, 16)]
            for t in range(1, 16):
                acc = acc + rows_a[t, pl.ds(k * 16, 16)]
            clamped = jnp.maximum(acc, 1e-8)
            am = jnp.where(acc > 1e-8, jnp.float32(AGG), jnp.float32(0.0))
            a_ref[pl.ds(j * 128 + k * 16, 16)] = 1.0 - am
            b_ref[pl.ds(j * 128 + k * 16, 16)] = am / clamped
        return 0
    lax.fori_loop(0, NODES_TILE // 128, _coef_chunk, 0)

    def _node_chunk(j, _):
        nb = nbase + j * NODE_CHUNK
        @pl.when(j > 0)
        def _():
            pltpu.make_async_copy(rows_b, out_h.at[pl.ds(coff + nb, NODE_CHUNK)],
                                  sem_sa).wait()
        pltpu.async_copy(enh_sh.at[pl.ds(nb, NODE_CHUNK)], rows_a, sem_ga)
        pltpu.async_copy(ftab.at[pl.ds(coff + nb, NODE_CHUNK)], rows_b, sem_gb)
        pltpu.make_async_copy(enh_sh.at[pl.ds(nb, NODE_CHUNK)], rows_a, sem_ga).wait()
        pltpu.make_async_copy(ftab.at[pl.ds(coff + nb, NODE_CHUNK)], rows_b, sem_gb).wait()

        def _blend(q, _):
            ag = a_ref[pl.ds(j * NODE_CHUNK + q * 16, 16)]
            bg = b_ref[pl.ds(j * NODE_CHUNK + q * 16, 16)]
            for l in range(16):
                i = q * 16 + l
                av = ag[l]
                bv = bg[l]
                for k in range(DH // 16):
                    rows_b[i, pl.ds(k * 16, 16)] = (
                        rows_b[i, pl.ds(k * 16, 16)] * av
                        + rows_a[i, pl.ds(k * 16, 16)] * bv
                    )
            return 0
        lax.fori_loop(0, NODE_CHUNK // 16, _blend, 0)

        pltpu.async_copy(rows_b, out_h.at[pl.ds(coff + nb, NODE_CHUNK)], sem_sa)
        return 0

    lax.fori_loop(0, N_NODE_CHUNKS, _node_chunk, 0)
    pltpu.make_async_copy(rows_b, out_h.at[pl.ds(coff + nbase, NODE_CHUNK)],
                          sem_sa).wait()


def kernel(features, edges, edge_weights):
    f0 = jnp.pad(features[:, :DH], ((0, N_PAD - N_NODES), (0, 0)))
    f1 = jnp.pad(features[:, DH:], ((0, N_PAD - N_NODES), (0, 0)))
    ftab = jnp.concatenate([f0, f1], axis=0)
    src = jnp.pad(edges[:, 0], (0, E_PAD - N_EDGES))
    tgt = jnp.pad(edges[:, 1], (0, E_PAD - N_EDGES))
    wb = jax.lax.bitcast_convert_type(
        jnp.pad(edge_weights, (0, E_PAD - N_EDGES)), jnp.int32)
    edata = jnp.stack(
        [src.reshape(-1, CHUNK), tgt.reshape(-1, CHUNK), wb.reshape(-1, CHUNK)],
        axis=1).reshape(-1)
    out = _sc_body(ftab, edata)
    return jnp.concatenate([out[:N_NODES], out[N_PAD:N_PAD + N_NODES]], axis=1)


# speedup vs baseline: 1.8129x; 1.1597x over previous
import functools

import jax
import jax.numpy as jnp
from jax import lax
from jax.experimental import pallas as pl
from jax.experimental.pallas import tpu as pltpu
from jax.experimental.pallas import tpu_sc as plsc

AGG = 0.3
N_NODES = 10000
N_PAD = 10240
D = 256
DH = 128
N_EDGES = 160000
E_PAD = 163840
E_TILE = E_PAD // 16
CHUNK = 64
BLOCK = 1024
N_BLOCKS = E_TILE // BLOCK
PAIRS = BLOCK // (2 * CHUNK)
NODES_TILE = N_PAD // 16
NODE_CHUNK = 64
N_NODE_CHUNKS = NODES_TILE // NODE_CHUNK

_mesh = plsc.VectorSubcoreMesh(core_axis_name="c", subcore_axis_name="s")


@functools.partial(
    pl.kernel,
    mesh=_mesh,
    out_type=jax.ShapeDtypeStruct((2 * N_PAD, DH), jnp.float32),
    scratch_types=[
        pltpu.VMEM((BLOCK,), jnp.int32),
        pltpu.VMEM((BLOCK,), jnp.int32),
        pltpu.VMEM((BLOCK,), jnp.float32),
        pltpu.VMEM((CHUNK,), jnp.int32),
        pltpu.VMEM((CHUNK,), jnp.int32),
        pltpu.VMEM((CHUNK,), jnp.int32),
        pltpu.VMEM((CHUNK,), jnp.int32),
        pltpu.VMEM((CHUNK, DH), jnp.float32),
        pltpu.VMEM((CHUNK, DH), jnp.float32),
        pltpu.VMEM((N_PAD,), jnp.float32),
        pltpu.VMEM((NODES_TILE,), jnp.float32),
        pltpu.VMEM((NODES_TILE,), jnp.float32),
        pltpu.VMEM_SHARED((N_PAD, DH), jnp.float32),
        pltpu.VMEM_SHARED((16, N_PAD), jnp.float32),
        pltpu.SemaphoreType.DMA,
        pltpu.SemaphoreType.DMA,
        pltpu.SemaphoreType.DMA,
        pltpu.SemaphoreType.DMA,
    ],
    compiler_params=pltpu.CompilerParams(needs_layout_passes=False),
)
def _sc_body(ftab, src_h, tgt_h, w_h, out_h,
             sblk, tblk, wblk, idx_a, tgt_a, idx_b, tgt_b,
             rows_a, rows_b, counts_v, a_ref, b_ref,
             enh_sh, counts_sh, sem_ga, sem_gb, sem_sa, sem_sb):
    c = lax.axis_index("c")
    s = lax.axis_index("s")
    zero16 = jnp.zeros((16,), jnp.float32)

    def _zc(i, _):
        counts_v[pl.ds(i * 16, 16)] = zero16
        return 0
    lax.fori_loop(0, N_PAD // 16, _zc, 0)

    def _zr(i, _):
        for k in range(DH // 16):
            rows_a[i, pl.ds(k * 16, 16)] = zero16
            rows_b[i, pl.ds(k * 16, 16)] = zero16
        return 0
    lax.fori_loop(0, CHUNK, _zr, 0)

    nbase = s * NODES_TILE
    def _zs(j, _):
        pltpu.sync_copy(rows_a, enh_sh.at[pl.ds(nbase + 2 * j * CHUNK, CHUNK)])
        pltpu.sync_copy(rows_b, enh_sh.at[pl.ds((2 * j + 1) * CHUNK + nbase, CHUNK)])
        return 0
    lax.fori_loop(0, NODES_TILE // (2 * CHUNK), _zs, 0)

    plsc.subcore_barrier()

    ebase = s * E_TILE
    coff = c * N_PAD

    def _prep(o, idx_x, tgt_x):
        for k in range(CHUNK // 16):
            idx_x[pl.ds(k * 16, 16)] = sblk[pl.ds(o + k * 16, 16)] + coff
            tgt_x[pl.ds(k * 16, 16)] = tblk[pl.ds(o + k * 16, 16)]

    def _scale_counts(o, rows_x):
        def _scale(q, _):
            wg = wblk[pl.ds(o + q * 16, 16)]
            for l in range(16):
                i = q * 16 + l
                ws = wg[l]
                for k in range(DH // 16):
                    rows_x[i, pl.ds(k * 16, 16)] = rows_x[i, pl.ds(k * 16, 16)] * ws
            return 0
        lax.fori_loop(0, CHUNK // 16, _scale, 0)
        for k in range(CHUNK // 16):
            plsc.addupdate_scatter(
                counts_v,
                [tblk[pl.ds(o + k * 16, 16)]],
                wblk[pl.ds(o + k * 16, 16)],
            )

    def _block(blk, _):
        bb = ebase + blk * BLOCK
        pltpu.sync_copy(src_h.at[pl.ds(bb, BLOCK)], sblk)
        pltpu.sync_copy(tgt_h.at[pl.ds(bb, BLOCK)], tblk)
        pltpu.sync_copy(w_h.at[pl.ds(bb, BLOCK)], wblk)

        @pl.when(blk > 0)
        def _():
            pltpu.make_async_copy(rows_a, enh_sh.at[tgt_a], sem_sa).wait()
        _prep(0, idx_a, tgt_a)
        pltpu.async_copy(ftab.at[idx_a], rows_a, sem_ga)

        def _pair(p, _):
            oa = p * (2 * CHUNK)
            ob = oa + CHUNK

            @pl.when(blk + p > 0)
            def _():
                pltpu.make_async_copy(rows_b, enh_sh.at[tgt_b], sem_sb).wait()
            _prep(ob, idx_b, tgt_b)
            pltpu.async_copy(ftab.at[idx_b], rows_b, sem_gb)
            pltpu.make_async_copy(ftab.at[idx_a], rows_a, sem_ga).wait()
            _scale_counts(oa, rows_a)
            pltpu.async_copy(rows_a, enh_sh.at[tgt_a], sem_sa, add=True)

            @pl.when(p < PAIRS - 1)
            def _():
                pltpu.make_async_copy(rows_a, enh_sh.at[tgt_a], sem_sa).wait()
                _prep(ob + CHUNK, idx_a, tgt_a)
                pltpu.async_copy(ftab.at[idx_a], rows_a, sem_ga)
            pltpu.make_async_copy(ftab.at[idx_b], rows_b, sem_gb).wait()
            _scale_counts(ob, rows_b)
            pltpu.async_copy(rows_b, enh_sh.at[tgt_b], sem_sb, add=True)
            return 0

        lax.fori_loop(0, PAIRS, _pair, 0)
        return 0

    lax.fori_loop(0, N_BLOCKS, _block, 0)

    pltpu.make_async_copy(rows_a, enh_sh.at[tgt_a], sem_sa).wait()
    pltpu.make_async_copy(rows_b, enh_sh.at[tgt_b], sem_sb).wait()

    pltpu.sync_copy(counts_v, counts_sh.at[s])
    plsc.subcore_barrier()

    c16 = rows_a.at[pl.ds(0, 16)]

    def _coef_chunk(j, _):
        pltpu.sync_copy(counts_sh.at[:, pl.ds(nbase + j * 128, 128)], c16)
        for k in range(128 // 16):
            acc = rows_a[0, pl.ds(k * 16, 16)]
            for t in range(1, 16):
                acc = acc + rows_a[t, pl.ds(k * 16, 16)]
            clamped = jnp.maximum(acc, 1e-8)
            am = jnp.where(acc > 1e-8, jnp.float32(AGG), jnp.float32(0.0))
            a_ref[pl.ds(j * 128 + k * 16, 16)] = 1.0 - am
            b_ref[pl.ds(j * 128 + k * 16, 16)] = am / clamped
        return 0
    lax.fori_loop(0, NODES_TILE // 128, _coef_chunk, 0)

    def _node_chunk(j, _):
        nb = nbase + j * NODE_CHUNK
        pltpu.sync_copy(enh_sh.at[pl.ds(nb, NODE_CHUNK)], rows_a)
        pltpu.sync_copy(ftab.at[pl.ds(coff + nb, NODE_CHUNK)], rows_b)

        def _blend(q, _):
            ag = a_ref[pl.ds(j * NODE_CHUNK + q * 16, 16)]
            bg = b_ref[pl.ds(j * NODE_CHUNK + q * 16, 16)]
            for l in range(16):
                i = q * 16 + l
                av = ag[l]
                bv = bg[l]
                for k in range(DH // 16):
                    rows_b[i, pl.ds(k * 16, 16)] = (
                        rows_b[i, pl.ds(k * 16, 16)] * av
                        + rows_a[i, pl.ds(k * 16, 16)] * bv
                    )
            return 0
        lax.fori_loop(0, NODE_CHUNK // 16, _blend, 0)

        pltpu.sync_copy(rows_b, out_h.at[pl.ds(coff + nb, NODE_CHUNK)])
        return 0

    lax.fori_loop(0, N_NODE_CHUNKS, _node_chunk, 0)


def kernel(features, edges, edge_weights):
    f0 = jnp.pad(features[:, :DH], ((0, N_PAD - N_NODES), (0, 0)))
    f1 = jnp.pad(features[:, DH:], ((0, N_PAD - N_NODES), (0, 0)))
    ftab = jnp.concatenate([f0, f1], axis=0)
    src = jnp.pad(edges[:, 0], (0, E_PAD - N_EDGES))
    tgt = jnp.pad(edges[:, 1], (0, E_PAD - N_EDGES))
    w = jnp.pad(edge_weights, (0, E_PAD - N_EDGES))
    out = _sc_body(ftab, src, tgt, w)
    return jnp.concatenate([out[:N_NODES], out[N_PAD:N_PAD + N_NODES]], axis=1)


# speedup vs baseline: 1.8272x; 1.0079x over previous
import functools

import jax
import jax.numpy as jnp
from jax import lax
from jax.experimental import pallas as pl
from jax.experimental.pallas import tpu as pltpu
from jax.experimental.pallas import tpu_sc as plsc

AGG = 0.3
N_NODES = 10000
N_PAD = 10240
D = 256
DH = 128
N_EDGES = 160000
E_PAD = 163840
E_TILE = E_PAD // 16
CHUNK = 64
BLOCK = 1024
N_BLOCKS = E_TILE // BLOCK
PAIRS = BLOCK // (2 * CHUNK)
NODES_TILE = N_PAD // 16
NODE_CHUNK = 64
N_NODE_CHUNKS = NODES_TILE // NODE_CHUNK

_mesh = plsc.VectorSubcoreMesh(core_axis_name="c", subcore_axis_name="s")


@functools.partial(
    pl.kernel,
    mesh=_mesh,
    out_type=jax.ShapeDtypeStruct((2 * N_PAD, DH), jnp.float32),
    scratch_types=[
        pltpu.VMEM((BLOCK,), jnp.int32),
        pltpu.VMEM((BLOCK,), jnp.int32),
        pltpu.VMEM((BLOCK,), jnp.float32),
        pltpu.VMEM((CHUNK,), jnp.int32),
        pltpu.VMEM((CHUNK,), jnp.int32),
        pltpu.VMEM((CHUNK,), jnp.int32),
        pltpu.VMEM((CHUNK,), jnp.int32),
        pltpu.VMEM((CHUNK, DH), jnp.float32),
        pltpu.VMEM((CHUNK, DH), jnp.float32),
        pltpu.VMEM((N_PAD,), jnp.float32),
        pltpu.VMEM((NODES_TILE,), jnp.float32),
        pltpu.VMEM((NODES_TILE,), jnp.float32),
        pltpu.VMEM_SHARED((N_PAD, DH), jnp.float32),
        pltpu.VMEM_SHARED((16, N_PAD), jnp.float32),
        pltpu.SemaphoreType.DMA,
        pltpu.SemaphoreType.DMA,
        pltpu.SemaphoreType.DMA,
        pltpu.SemaphoreType.DMA,
    ],
    compiler_params=pltpu.CompilerParams(needs_layout_passes=False),
)
def _sc_body(ftab, src_h, tgt_h, w_h, out_h,
             sblk, tblk, wblk, idx_a, tgt_a, idx_b, tgt_b,
             rows_a, rows_b, counts_v, a_ref, b_ref,
             enh_sh, counts_sh, sem_ga, sem_gb, sem_sa, sem_sb):
    c = lax.axis_index("c")
    s = lax.axis_index("s")
    zero16 = jnp.zeros((16,), jnp.float32)

    def _zc(i, _):
        counts_v[pl.ds(i * 16, 16)] = zero16
        return 0
    lax.fori_loop(0, N_PAD // 16, _zc, 0)

    def _zr(i, _):
        for k in range(DH // 16):
            rows_a[i, pl.ds(k * 16, 16)] = zero16
            rows_b[i, pl.ds(k * 16, 16)] = zero16
        return 0
    lax.fori_loop(0, CHUNK, _zr, 0)

    nbase = s * NODES_TILE
    def _zs(j, _):
        pltpu.sync_copy(rows_a, enh_sh.at[pl.ds(nbase + 2 * j * CHUNK, CHUNK)])
        pltpu.sync_copy(rows_b, enh_sh.at[pl.ds((2 * j + 1) * CHUNK + nbase, CHUNK)])
        return 0
    lax.fori_loop(0, NODES_TILE // (2 * CHUNK), _zs, 0)

    plsc.subcore_barrier()

    ebase = s * E_TILE
    coff = c * N_PAD

    def _prep(o, idx_x, tgt_x):
        for k in range(CHUNK // 16):
            idx_x[pl.ds(k * 16, 16)] = sblk[pl.ds(o + k * 16, 16)] + coff
            tgt_x[pl.ds(k * 16, 16)] = tblk[pl.ds(o + k * 16, 16)]

    def _scale_counts(o, rows_x):
        def _scale(q, _):
            wg = wblk[pl.ds(o + q * 16, 16)]
            for l in range(16):
                i = q * 16 + l
                ws = wg[l]
                for k in range(DH // 16):
                    rows_x[i, pl.ds(k * 16, 16)] = rows_x[i, pl.ds(k * 16, 16)] * ws
            return 0
        lax.fori_loop(0, CHUNK // 16, _scale, 0)
        for k in range(CHUNK // 16):
            plsc.addupdate_scatter(
                counts_v,
                [tblk[pl.ds(o + k * 16, 16)]],
                wblk[pl.ds(o + k * 16, 16)],
            )

    def _block(blk, _):
        bb = ebase + blk * BLOCK
        pltpu.sync_copy(src_h.at[pl.ds(bb, BLOCK)], sblk)
        pltpu.sync_copy(tgt_h.at[pl.ds(bb, BLOCK)], tblk)
        pltpu.sync_copy(w_h.at[pl.ds(bb, BLOCK)], wblk)

        @pl.when(blk > 0)
        def _():
            pltpu.make_async_copy(rows_a, enh_sh.at[tgt_a], sem_sa).wait()
        _prep(0, idx_a, tgt_a)
        pltpu.async_copy(ftab.at[idx_a], rows_a, sem_ga)

        def _pair(p, _):
            oa = p * (2 * CHUNK)
            ob = oa + CHUNK

            @pl.when(blk + p > 0)
            def _():
                pltpu.make_async_copy(rows_b, enh_sh.at[tgt_b], sem_sb).wait()
            _prep(ob, idx_b, tgt_b)
            pltpu.async_copy(ftab.at[idx_b], rows_b, sem_gb)
            pltpu.make_async_copy(ftab.at[idx_a], rows_a, sem_ga).wait()
            _scale_counts(oa, rows_a)
            pltpu.async_copy(rows_a, enh_sh.at[tgt_a], sem_sa, add=True)

            @pl.when(p < PAIRS - 1)
            def _():
                pltpu.make_async_copy(rows_a, enh_sh.at[tgt_a], sem_sa).wait()
                _prep(ob + CHUNK, idx_a, tgt_a)
                pltpu.async_copy(ftab.at[idx_a], rows_a, sem_ga)
            pltpu.make_async_copy(ftab.at[idx_b], rows_b, sem_gb).wait()
            _scale_counts(ob, rows_b)
            pltpu.async_copy(rows_b, enh_sh.at[tgt_b], sem_sb, add=True)
            return 0

        lax.fori_loop(0, PAIRS, _pair, 0)
        return 0

    lax.fori_loop(0, N_BLOCKS, _block, 0)

    pltpu.make_async_copy(rows_a, enh_sh.at[tgt_a], sem_sa).wait()
    pltpu.make_async_copy(rows_b, enh_sh.at[tgt_b], sem_sb).wait()

    pltpu.sync_copy(counts_v, counts_sh.at[s])
    plsc.subcore_barrier()

    c16 = rows_a.at[pl.ds(0, 16)]

    def _coef_chunk(j, _):
        pltpu.sync_copy(counts_sh.at[:, pl.ds(nbase + j * 128, 128)], c16)
        for k in range(128 // 16):
            acc = rows_a[0, pl.ds(k * 16, 16)]
            for t in range(1, 16):
                acc = acc + rows_a[t, pl.ds(k * 16, 16)]
            clamped = jnp.maximum(acc, 1e-8)
            am = jnp.where(acc > 1e-8, jnp.float32(AGG), jnp.float32(0.0))
            a_ref[pl.ds(j * 128 + k * 16, 16)] = 1.0 - am
            b_ref[pl.ds(j * 128 + k * 16, 16)] = am / clamped
        return 0
    lax.fori_loop(0, NODES_TILE // 128, _coef_chunk, 0)

    def _node_chunk(j, _):
        nb = nbase + j * NODE_CHUNK
        @pl.when(j > 0)
        def _():
            pltpu.make_async_copy(rows_b, out_h.at[pl.ds(coff + nb, NODE_CHUNK)],
                                  sem_sa).wait()
        pltpu.async_copy(enh_sh.at[pl.ds(nb, NODE_CHUNK)], rows_a, sem_ga)
        pltpu.async_copy(ftab.at[pl.ds(coff + nb, NODE_CHUNK)], rows_b, sem_gb)
        pltpu.make_async_copy(enh_sh.at[pl.ds(nb, NODE_CHUNK)], rows_a, sem_ga).wait()
        pltpu.make_async_copy(ftab.at[pl.ds(coff + nb, NODE_CHUNK)], rows_b, sem_gb).wait()

        def _blend(q, _):
            ag = a_ref[pl.ds(j * NODE_CHUNK + q * 16, 16)]
            bg = b_ref[pl.ds(j * NODE_CHUNK + q * 16, 16)]
            for l in range(16):
                i = q * 16 + l
                av = ag[l]
                bv = bg[l]
                for k in range(DH // 16):
                    rows_b[i, pl.ds(k * 16, 16)] = (
                        rows_b[i, pl.ds(k * 16, 16)] * av
                        + rows_a[i, pl.ds(k * 16, 16)] * bv
                    )
            return 0
        lax.fori_loop(0, NODE_CHUNK // 16, _blend, 0)

        pltpu.async_copy(rows_b, out_h.at[pl.ds(coff + nb, NODE_CHUNK)], sem_sa)
        return 0

    lax.fori_loop(0, N_NODE_CHUNKS, _node_chunk, 0)
    pltpu.make_async_copy(rows_b, out_h.at[pl.ds(coff + nbase, NODE_CHUNK)],
                          sem_sa).wait()


def kernel(features, edges, edge_weights):
    f0 = jnp.pad(features[:, :DH], ((0, N_PAD - N_NODES), (0, 0)))
    f1 = jnp.pad(features[:, DH:], ((0, N_PAD - N_NODES), (0, 0)))
    ftab = jnp.concatenate([f0, f1], axis=0)
    src = jnp.pad(edges[:, 0], (0, E_PAD - N_EDGES))
    tgt = jnp.pad(edges[:, 1], (0, E_PAD - N_EDGES))
    w = jnp.pad(edge_weights, (0, E_PAD - N_EDGES))
    out = _sc_body(ftab, src, tgt, w)
    return jnp.concatenate([out[:N_NODES], out[N_PAD:N_PAD + N_NODES]], axis=1)
